# baseline (speedup 1.0000x reference)
import jax
import jax.numpy as jnp
from jax import lax
from jax.experimental import pallas as pl
from jax.experimental.pallas import tpu as pltpu
from jax.experimental.pallas import tpu_sc as plsc

_NC = 2
_NS = 16
_NW = _NC * _NS

_B = 4096
_L = 50
_LP = 56
_D = 128
_RPW = _B // _NW
_R = 2
_CH = _RPW // _R
_NBUF = 3


def _sc_body(ids_hbm, emb_hbm, glove_hbm, e_out, g_out, idx_all, ebuf, gbuf,
             *sems):
    gsems = sems[:_NBUF]
    ssems = sems[_NBUF:]
    wid = lax.axis_index("s") * _NC + lax.axis_index("c")
    r0w = wid * _RPW

    pltpu.sync_copy(ids_hbm.at[pl.ds(r0w, _RPW), :], idx_all)

    def start_gathers(c, b):
        for r in range(_R):
            pltpu.async_copy(emb_hbm.at[idx_all.at[c * _R + r]],
                             ebuf.at[b, r, pl.ds(0, _L)], gsems[b])
            pltpu.async_copy(glove_hbm.at[idx_all.at[c * _R + r]],
                             gbuf.at[b, r, pl.ds(0, _L)], gsems[b])

    def wait_gathers(b):
        for r in range(_R):
            pltpu.make_async_copy(emb_hbm.at[idx_all.at[0]],
                                  ebuf.at[b, r, pl.ds(0, _L)],
                                  gsems[b]).wait()
            pltpu.make_async_copy(glove_hbm.at[idx_all.at[0]],
                                  gbuf.at[b, r, pl.ds(0, _L)],
                                  gsems[b]).wait()

    def store_dst(out, c):
        return out.at[pl.ds(r0w + c * _R, _R)]

    def start_stores(c, b):
        pltpu.async_copy(ebuf.at[b], store_dst(e_out, c), ssems[b])
        pltpu.async_copy(gbuf.at[b], store_dst(g_out, c), ssems[b])

    def wait_stores(c, b):
        pltpu.make_async_copy(ebuf.at[b], store_dst(e_out, c), ssems[b]).wait()
        pltpu.make_async_copy(gbuf.at[b], store_dst(g_out, c), ssems[b]).wait()

    for c in range(_CH):
        b = c % _NBUF
        if c >= _NBUF:
            wait_stores(c - _NBUF, b)
        start_gathers(c, b)
        if c >= 2:
            wait_gathers((c - 2) % _NBUF)
            start_stores(c - 2, (c - 2) % _NBUF)
    for c in range(_CH - 2, _CH):
        wait_gathers(c % _NBUF)
        start_stores(c, c % _NBUF)
    for c in range(_CH - _NBUF, _CH):
        wait_stores(c, c % _NBUF)


def _pack_body(e_ref, g_ref, o_ref):
    o_ref[...] = jnp.concatenate([jnp.tanh(e_ref[...]), g_ref[...]], axis=-1)


_TC_ROWS = 256


@jax.jit
def _run(ids, emb_table, glove_table):
    mesh = plsc.VectorSubcoreMesh(
        core_axis_name="c", subcore_axis_name="s",
        num_cores=_NC, num_subcores=_NS)
    gather = pl.kernel(
        _sc_body,
        out_type=(
            jax.ShapeDtypeStruct((_B, _LP, _D), jnp.float32),
            jax.ShapeDtypeStruct((_B, _LP, _D), jnp.float32),
        ),
        mesh=mesh,
        scratch_types=(
            [pltpu.VMEM((_RPW, _L), jnp.int32),
             pltpu.VMEM((_NBUF, _R, _LP, _D), jnp.float32),
             pltpu.VMEM((_NBUF, _R, _LP, _D), jnp.float32)]
            + [pltpu.SemaphoreType.DMA] * (2 * _NBUF)
        ),
    )
    e_out, g_out = gather(ids, emb_table, glove_table)

    pack = pl.pallas_call(
        _pack_body,
        grid=(_B // _TC_ROWS, _LP // 8),
        in_specs=[
            pl.BlockSpec((_TC_ROWS, 8, _D), lambda i, j: (i, j, 0)),
            pl.BlockSpec((_TC_ROWS, 8, _D), lambda i, j: (i, j, 0)),
        ],
        out_specs=pl.BlockSpec((_TC_ROWS, 8, 2 * _D), lambda i, j: (i, j, 0)),
        out_shape=jax.ShapeDtypeStruct((_B, _L, 2 * _D), jnp.float32),
    )
    return pack(e_out, g_out)


def kernel(input_ids, emb_table, glove_table):
    return _run(input_ids.astype(jnp.int32), emb_table, glove_table)
